# baseline (speedup 1.0000x reference)
import functools

import jax
import jax.numpy as jnp
from jax import lax
from jax.experimental import pallas as pl
from jax.experimental.pallas import tpu as pltpu
from jax.experimental.pallas import tpu_sc as plsc

F32 = jnp.float32
_NC, _NS, _NW, _L = 2, 16, 32, 16
_C = 400
_CS = 200
_NCH = 12800
_ZR = 832


def _mesh():
    return plsc.VectorSubcoreMesh(core_axis_name="c", subcore_axis_name="s")


def _gpad(e):
    return {10000: 12800, 20000: 25600, 40000: 44800,
            80000: 83200, 200000: 204800}[e]


def _sc_gather2(ts, tr, src_p, dst_p):
    e_pad = src_p.shape[0]
    c2 = 200
    cpt = e_pad // (_NW * c2)
    span = cpt * c2

    @functools.partial(
        pl.kernel,
        out_type=jax.ShapeDtypeStruct((e_pad, 128), F32),
        mesh=_mesh(),
        scratch_types=[
            pltpu.VMEM((span,), jnp.int32),
            pltpu.VMEM((span,), jnp.int32),
            pltpu.VMEM((c2, 128), F32),
            pltpu.VMEM((c2, 128), F32),
            pltpu.SemaphoreType.DMA,
            pltpu.SemaphoreType.DMA,
            pltpu.SemaphoreType.DMA,
            pltpu.SemaphoreType.DMA,
            pltpu.SemaphoreType.DMA,
            pltpu.SemaphoreType.DMA,
        ],
    )
    def k(ts_hbm, tr_hbm, src_hbm, dst_hbm, out_hbm, isb, idb,
          a0, a1, sa0, sa1, sb0, sb1, so0, so1):
        av = (a0, a1)
        sa = (sa0, sa1)
        sb = (sb0, sb1)
        so = (so0, so1)
        wid = lax.axis_index("s") * _NC + lax.axis_index("c")
        wbase = wid * span
        pltpu.sync_copy(src_hbm.at[pl.ds(wbase, span)], isb)
        pltpu.sync_copy(dst_hbm.at[pl.ds(wbase, span)], idb)
        for j in range(cpt):
            b = j % 2
            if j >= 2:
                pltpu.make_async_copy(
                    av[b], out_hbm.at[pl.ds(wbase + (j - 2) * c2, c2)],
                    so[b]).wait()
            pltpu.async_copy(ts_hbm.at[isb.at[pl.ds(j * c2, c2)]], av[b],
                             sa[b])
            pltpu.make_async_copy(
                ts_hbm.at[isb.at[pl.ds(j * c2, c2)]], av[b], sa[b]).wait()
            pltpu.async_copy(tr_hbm.at[idb.at[pl.ds(j * c2, c2)]], av[b],
                             sb[b], add=True)
            pltpu.make_async_copy(
                tr_hbm.at[idb.at[pl.ds(j * c2, c2)]], av[b], sb[b]).wait()
            pltpu.async_copy(av[b], out_hbm.at[pl.ds(wbase + j * c2, c2)],
                             so[b])

        for j in range(max(0, cpt - 2), cpt):
            b = j % 2
            pltpu.make_async_copy(
                av[b], out_hbm.at[pl.ds(wbase + j * c2, c2)], so[b]).wait()

    return k(ts, tr, src_p, dst_p)


def _sc_scatter2(msgs, idx, n_pad, zeros):
    e_tot = msgs.shape[0]
    n_chunks = e_tot // _C
    iters = -(-n_chunks // _NW)
    share = n_pad // _NS

    @functools.partial(
        pl.kernel,
        out_type=jax.ShapeDtypeStruct((2, n_pad, 128), F32),
        mesh=_mesh(),
        scratch_types=[
            pltpu.VMEM((_C,), jnp.int32),
            pltpu.VMEM((_C,), jnp.int32),
            pltpu.VMEM((_C, 128), F32),
            pltpu.VMEM((_C, 128), F32),
            pltpu.VMEM_SHARED((n_pad, 128), F32),
            pltpu.SemaphoreType.DMA,
            pltpu.SemaphoreType.DMA,
            pltpu.SemaphoreType.DMA,
            pltpu.SemaphoreType.DMA,
        ],
    )
    def k(msgs_hbm, idx_hbm, zeros_hbm, out_hbm, iv0, iv1, rv0, rv1, acc,
          sm0, sm1, ss0, ss1):
        iv = (iv0, iv1)
        rv = (rv0, rv1)
        cid = lax.axis_index("c")
        sid = lax.axis_index("s")
        wid = sid * _NC + cid
        sm = (sm0, sm1)
        ss = (ss0, ss1)
        pltpu.sync_copy(zeros_hbm.at[pl.ds(0, share)],
                        acc.at[pl.ds(sid * share, share)])
        plsc.subcore_barrier()
        for j in range(iters):
            g = j * _NW + wid
            b = j % 2
            if j >= 2:

                @pl.when((j - 2) * _NW + wid < n_chunks)
                def _():
                    pltpu.make_async_copy(rv[b], acc.at[iv[b]],
                                          ss[b]).wait()

            @pl.when(g < n_chunks)
            def _():
                base = g * _C
                pltpu.sync_copy(idx_hbm.at[pl.ds(base, _C)], iv[b])
                pltpu.async_copy(msgs_hbm.at[pl.ds(base, _C)], rv[b],
                                 sm[b]).wait()
                pltpu.async_copy(rv[b], acc.at[iv[b]], ss[b],
                                 add=True)

        for j in range(max(0, iters - 2), iters):
            b = j % 2

            @pl.when(j * _NW + wid < n_chunks)
            def _():
                pltpu.make_async_copy(rv[b], acc.at[iv[b]],
                                      ss[b]).wait()

        plsc.subcore_barrier()
        pltpu.sync_copy(acc.at[pl.ds(sid * share, share)],
                        out_hbm.at[cid, pl.ds(sid * share, share)])

    return k(msgs, idx, zeros)


def _sc_scatter_chunked(msgs, lidx, zeros, nch, nq):
    e_tot = msgs.shape[0]
    cs = 200
    n_chunks = e_tot // cs
    iters = -(-n_chunks // _NS)
    acc_rows = nch + 128
    share_z = acc_rows // _NS
    share_w = nch // _NS

    @functools.partial(
        pl.kernel,
        out_type=jax.ShapeDtypeStruct((nq * nch, 128), F32),
        mesh=_mesh(),
        scratch_types=[
            pltpu.VMEM((cs,), jnp.int32),
            pltpu.VMEM((cs,), jnp.int32),
            pltpu.VMEM((cs, 128), F32),
            pltpu.VMEM_SHARED((acc_rows, 128), F32),
            pltpu.SemaphoreType.DMA,
            pltpu.SemaphoreType.DMA,
            pltpu.SemaphoreType.DMA,
            pltpu.SemaphoreType.DMA,
            pltpu.SemaphoreType.DMA,
            pltpu.SemaphoreType.DMA,
        ],
    )
    def k(msgs_hbm, lidx_hbm, zeros_hbm, out_hbm, iv0, iv1, rv0, acc,
          si0, si1, sm0, sm1, ss0, ss1):
        iv = (iv0, iv1)
        rv = (rv0, rv0)
        si = (si0, si1)
        sm = (sm0, sm1)
        ss = (ss0, ss1)
        cid = lax.axis_index("c")
        sid = lax.axis_index("s")

        def chunk_of(j):
            return j * _NS + sid

        for p in range(nq // 2):
            chunk_id = 2 * p + cid
            ibase = chunk_id * e_tot
            pltpu.sync_copy(zeros_hbm.at[pl.ds(0, share_z)],
                            acc.at[pl.ds(sid * share_z, share_z)])
            plsc.subcore_barrier()

            @pl.when(chunk_of(0) < n_chunks)
            def _():
                pltpu.async_copy(
                    lidx_hbm.at[pl.ds(ibase + chunk_of(0) * cs, cs)],
                    iv[0], si[0])

            for j in range(iters):
                g = chunk_of(j)
                b = j % 2
                nb = (j + 1) % 2

                if j >= 1:

                    @pl.when(chunk_of(j - 1) < n_chunks)
                    def _():
                        pltpu.make_async_copy(rv[nb], acc.at[iv[nb]],
                                              ss[nb]).wait()

                @pl.when(g < n_chunks)
                def _():
                    pltpu.async_copy(msgs_hbm.at[pl.ds(g * cs, cs)], rv[b],
                                     sm[b])

                if j + 1 < iters:

                    @pl.when(chunk_of(j + 1) < n_chunks)
                    def _():
                        pltpu.async_copy(
                            lidx_hbm.at[pl.ds(ibase + chunk_of(j + 1) * cs,
                                              cs)], iv[nb], si[nb])

                @pl.when(g < n_chunks)
                def _():
                    pltpu.make_async_copy(
                        lidx_hbm.at[pl.ds(ibase + g * cs, cs)], iv[b],
                        si[b]).wait()
                    pltpu.make_async_copy(msgs_hbm.at[pl.ds(g * cs, cs)],
                                          rv[b], sm[b]).wait()
                    pltpu.async_copy(rv[b], acc.at[iv[b]], ss[b], add=True)

            @pl.when(chunk_of(iters - 1) < n_chunks)
            def _():
                b = (iters - 1) % 2
                pltpu.make_async_copy(rv[b], acc.at[iv[b]], ss[b]).wait()

            plsc.subcore_barrier()
            pltpu.sync_copy(
                acc.at[pl.ds(sid * share_w, share_w)],
                out_hbm.at[pl.ds(chunk_id * nch + sid * share_w, share_w)])
            plsc.subcore_barrier()

    return k(msgs, lidx, zeros)


def _bf16(v):
    return v.astype(jnp.bfloat16)


def _pick_b(n):
    for b in (2000, 640):
        if n % b == 0:
            return b
    raise ValueError(f"no row block for {n}")


def _rowspec(b):
    return pl.BlockSpec((b, 128), lambda i: (i, 0))


def _wspec(shape):
    return pl.BlockSpec(shape, lambda i: (0,) * len(shape))


def _ln(y):
    m = jnp.mean(y, axis=-1, keepdims=True)
    d = y - m
    v = jnp.mean(d * d, axis=-1, keepdims=True)
    return d * lax.rsqrt(v + 1e-5)


def _mm(x, w, b=None):
    n = x.shape[0]
    blk = _pick_b(n)
    has_b = b is not None

    def body(x_ref, w_ref, *rest):
        if has_b:
            b_ref, o_ref = rest
        else:
            (o_ref,) = rest
        acc = jnp.dot(_bf16(x_ref[...]), _bf16(w_ref[...]),
                      preferred_element_type=F32)
        if has_b:
            acc = acc + b_ref[...]
        o_ref[...] = acc

    args = [x, w] + ([b.reshape(1, 128)] if has_b else [])
    specs = [_rowspec(blk), _wspec((128, 128))] + ([_wspec((1, 128))] if has_b else [])
    return pl.pallas_call(
        body, grid=(n // blk,), in_specs=specs, out_specs=_rowspec(blk),
        out_shape=jax.ShapeDtypeStruct((n, 128), F32))(*args)


def _mlp2_ln(x, w1, b1, w2, b2):
    n = x.shape[0]
    blk = _pick_b(n)

    def body(x_ref, w1_ref, b1_ref, w2_ref, b2_ref, o_ref):
        h = (jnp.dot(_bf16(x_ref[...]), _bf16(w1_ref[...]),
                     preferred_element_type=F32) + b1_ref[...])
        h = jax.nn.silu(h)
        y = (jnp.dot(_bf16(h), _bf16(w2_ref[...]),
                     preferred_element_type=F32) + b2_ref[...])
        o_ref[...] = _ln(y)

    return pl.pallas_call(
        body, grid=(n // blk,),
        in_specs=[_rowspec(blk), _wspec((128, 128)), _wspec((1, 128)),
                  _wspec((128, 128)), _wspec((1, 128))],
        out_specs=_rowspec(blk),
        out_shape=jax.ShapeDtypeStruct((n, 128), F32),
    )(x, w1, b1.reshape(1, 128), w2, b2.reshape(1, 128))


def _edge_msgs(edge, gsum, we, w2, b2, update_edges):
    e_tot = edge.shape[0]
    blk = _pick_b(e_tot)

    def body(e_ref, gsum_ref, we_ref, w2_ref, b2_ref, m_ref, *ne):
        h = (jnp.dot(_bf16(e_ref[...]), _bf16(we_ref[...]),
                     preferred_element_type=F32)
             + gsum_ref[...])
        h = jax.nn.silu(h)
        y = jnp.dot(_bf16(h), _bf16(w2_ref[...]),
                    preferred_element_type=F32) + b2_ref[...]
        msg = _ln(y)
        m_ref[...] = msg
        if update_edges:
            ne[0][...] = e_ref[...] + msg

    out_shape = [jax.ShapeDtypeStruct((e_tot, 128), F32)]
    out_specs = [_rowspec(blk)]
    if update_edges:
        out_shape.append(jax.ShapeDtypeStruct((e_tot, 128), F32))
        out_specs.append(_rowspec(blk))
    res = pl.pallas_call(
        body, grid=(e_tot // blk,),
        in_specs=[_rowspec(blk), _rowspec(blk),
                  _wspec((128, 128)), _wspec((128, 128)), _wspec((1, 128))],
        out_specs=out_specs, out_shape=out_shape,
    )(edge, gsum, we, w2, b2.reshape(1, 128))
    return res if update_edges else (res[0], None)


def _node_update(rec, agg, wnr, wna, bn1, wn2, bn2, base_is_aggr, skip):
    n = rec.shape[0]
    blk = _pick_b(n)
    has_skip = skip is not None
    two_part = isinstance(agg, (tuple, list))

    def body(rec_ref, *refs):
        refs = list(refs)
        a0_ref = refs.pop(0)
        a1_ref = refs.pop(0) if two_part else None
        wnr_ref, wna_ref, bn1_ref, wn2_ref, bn2_ref = refs[:5]
        rest = refs[5:]
        if has_skip:
            skip_ref, o_ref = rest
        else:
            (o_ref,) = rest
        aggr = a0_ref[...]
        if two_part:
            aggr = aggr + a1_ref[...]
        h = (jnp.dot(_bf16(rec_ref[...]), _bf16(wnr_ref[...]),
                     preferred_element_type=F32)
             + jnp.dot(_bf16(aggr), _bf16(wna_ref[...]),
                       preferred_element_type=F32)
             + bn1_ref[...])
        h = jax.nn.silu(h)
        y = jnp.dot(_bf16(h), _bf16(wn2_ref[...]),
                    preferred_element_type=F32) + bn2_ref[...]
        upd = _ln(y)
        out = (aggr if base_is_aggr else rec_ref[...]) + upd
        if has_skip:
            out = out + skip_ref[...]
        o_ref[...] = out

    aggs = list(agg) if two_part else [agg]
    args = [rec] + aggs + [wnr, wna, bn1.reshape(1, 128), wn2,
                           bn2.reshape(1, 128)]
    specs = ([_rowspec(blk)] + [_rowspec(blk)] * len(aggs)
             + [_wspec((128, 128)), _wspec((128, 128)), _wspec((1, 128)),
                _wspec((128, 128)), _wspec((1, 128))])
    if has_skip:
        args.append(skip)
        specs.append(_rowspec(blk))
    return pl.pallas_call(
        body, grid=(n // blk,), in_specs=specs, out_specs=_rowspec(blk),
        out_shape=jax.ShapeDtypeStruct((n, 128), F32))(*args)


def _localize(dst, nch, nq):
    e_tot = dst.shape[0]
    blk = 2000
    d2 = dst.reshape(e_tot // blk, 1, blk)

    def body(d_ref, o_ref):
        v = d_ref[...]
        for q in range(nq):
            t = v - q * nch
            ok = (t >= 0) & (t < nch)
            o_ref[q, ...] = jnp.where(ok, t, nch)

    out = pl.pallas_call(
        body, grid=(e_tot // blk,),
        in_specs=[pl.BlockSpec((1, 1, blk), lambda i: (i, 0, 0))],
        out_specs=pl.BlockSpec((nq, 1, 1, blk), lambda i: (0, i, 0, 0)),
        out_shape=jax.ShapeDtypeStruct((nq, e_tot // blk, 1, blk), jnp.int32),
    )(d2)
    return out.reshape(nq * e_tot)


def _softplus(x):
    return jnp.maximum(x, 0.0) + jnp.log1p(jnp.exp(-jnp.abs(x)))


def _m2g_final(rec, aggr, wnr, wna, bn1, wn2, bn2, p1, pb1, w_mean, b_mean,
               w_std, b_std):
    n = rec.shape[0]
    blk = _pick_b(n)

    def body(rec_ref, a_ref, wnr_ref, wna_ref, bn1_ref, wn2_ref, bn2_ref,
             p1_ref, pb1_ref, wm_ref, bm_ref, ws_ref, bs_ref,
             mean_ref, std_ref):
        aggr = a_ref[...]
        h = (jnp.dot(_bf16(rec_ref[...]), _bf16(wnr_ref[...]),
                     preferred_element_type=F32)
             + jnp.dot(_bf16(aggr), _bf16(wna_ref[...]),
                       preferred_element_type=F32)
             + bn1_ref[...])
        h = jax.nn.silu(h)
        y = jnp.dot(_bf16(h), _bf16(wn2_ref[...]),
                    preferred_element_type=F32) + bn2_ref[...]
        grid_rep = aggr + _ln(y)
        g = jax.nn.silu(
            jnp.dot(grid_rep, p1_ref[...], preferred_element_type=F32)
            + pb1_ref[...])
        mean_ref[...] = (jnp.dot(g, wm_ref[...], preferred_element_type=F32)
                         + bm_ref[...])
        std_ref[...] = _softplus(
            jnp.dot(g, ws_ref[...], preferred_element_type=F32) + bs_ref[...])

    ospec = pl.BlockSpec((blk, 17), lambda i: (i, 0))
    return pl.pallas_call(
        body, grid=(n // blk,),
        in_specs=[_rowspec(blk), _rowspec(blk), _wspec((128, 128)),
                  _wspec((128, 128)), _wspec((1, 128)), _wspec((128, 128)),
                  _wspec((1, 128)), _wspec((128, 128)), _wspec((1, 128)),
                  _wspec((128, 17)), _wspec((1, 17)), _wspec((128, 17)),
                  _wspec((1, 17))],
        out_specs=[ospec, ospec],
        out_shape=[jax.ShapeDtypeStruct((n, 17), F32),
                   jax.ShapeDtypeStruct((n, 17), F32)],
    )(rec, aggr, wnr, wna, bn1.reshape(1, 128), wn2, bn2.reshape(1, 128),
      p1, pb1.reshape(1, 128), w_mean, b_mean.reshape(1, 17), w_std,
      b_std.reshape(1, 17))


def _padn(x, n):
    return jnp.pad(x, ((0, n - x.shape[0]), (0, 0)))


def _pad_idx(idx):
    return jnp.pad(idx, (0, _gpad(idx.shape[0]) - idx.shape[0]))


def _split_edge_w(ip):
    (w1, b1), (w2, b2) = ip["edge"]
    return w1[:128], w1[128:256], w1[256:384], b1, w2, b2


def _split_node_w(ip):
    (wn1, bn1), (wn2, bn2) = ip["node"]
    return wn1[:128], wn1[128:], bn1, wn2, bn2


def _stage(x_send, x_rec, edge, src, dst, ip, n_pad, zeros, *,
           base_is_aggr, skip, update_edges):
    we, ws, wr, b1, w2, b2 = _split_edge_w(ip)
    wnr, wna, bn1, wn2, bn2 = _split_node_w(ip)
    ts = _mm(x_send, ws)
    tr = _mm(x_rec, wr, b1)
    gsum = _sc_gather2(ts, tr, _pad_idx(src), _pad_idx(dst))
    msgs, new_edge = _edge_msgs(edge, gsum, we, w2, b2, update_edges)
    if n_pad <= 2560:
        agg = _sc_scatter2(msgs, dst, n_pad, zeros)
        agg = (agg[0], agg[1])
    else:
        nch = n_pad // 2
        lidx = _localize(dst, nch, 2)
        agg = _sc_scatter_chunked(msgs, lidx, zeros, nch, 2)
    x_new = _node_update(x_rec, agg, wnr, wna, bn1, wn2, bn2,
                         base_is_aggr, skip)
    return x_new, new_edge


def kernel(latent_samples, skip_in_0, skip_in_1, skip_in_2, skip_up_0,
           skip_up_1, mesh_emb_0, mesh_emb_1, mesh_emb_2, mesh_down_emb_0,
           mesh_down_emb_1, m2m_emb_0, m2m_emb_1, m2m_emb_2, m2g_emb,
           grid_static_features_hr, params, m2g_src, m2g_dst, md0_src,
           md0_dst, md1_src, md1_dst, m2m0, m2m1):
    zeros = jnp.zeros((_ZR, 128), F32)

    lat = _padn(latent_samples[0], 640)
    mesh1 = _padn(mesh_emb_1[0], 2560)
    mesh0 = _padn(mesh_emb_0[0], 10240)
    sk_up0 = _padn(skip_up_0[0], 2560)
    sk_in1 = _padn(skip_in_1[0], 2560)
    sk_in0 = _padn(skip_in_0[0], 10240)
    grid_static = grid_static_features_hr[0]

    (w1, b1), (w2, b2) = params["latent_embedder"]
    rep2 = _mlp2_ln(lat, w1, b1, w2, b2)

    rep1, _ = _stage(rep2, mesh1, mesh_down_emb_1[0], md1_src, md1_dst,
                     params["mesh_down"][1], 2560, zeros,
                     base_is_aggr=True, skip=sk_up0, update_edges=False)

    x, e = rep1, m2m_emb_1[0]
    x, e = _stage(x, x, e, m2m1[0], m2m1[1], params["intra_down"][1][0],
                  2560, zeros, base_is_aggr=False, skip=None,
                  update_edges=True)
    x, _ = _stage(x, x, e, m2m1[0], m2m1[1], params["intra_down"][1][1],
                  2560, zeros, base_is_aggr=False, skip=sk_in1,
                  update_edges=False)

    mesh_new0, _ = _stage(x, mesh0, mesh_down_emb_0[0], md0_src, md0_dst,
                          params["mesh_down"][0], 10240, zeros,
                          base_is_aggr=True, skip=None, update_edges=False)

    x, e = mesh_new0, m2m_emb_0[0]
    x, e = _stage(x, x, e, m2m0[0], m2m0[1], params["intra_down"][0][0],
                  10240, zeros, base_is_aggr=False, skip=None,
                  update_edges=True)
    x, _ = _stage(x, x, e, m2m0[0], m2m0[1], params["intra_down"][0][1],
                  10240, zeros, base_is_aggr=False, skip=sk_in0,
                  update_edges=False)

    ip = params["m2g"]
    we, ws, wr, eb1, w2, b2 = _split_edge_w(ip)
    wnr, wna, bn1, wn2, bn2 = _split_node_w(ip)
    ts = _mm(x, ws)
    tr = _mm(grid_static, wr, eb1)
    gsum = _sc_gather2(ts, tr, _pad_idx(m2g_src), _pad_idx(m2g_dst))
    msgs, _ = _edge_msgs(m2g_emb[0], gsum, we, w2, b2, False)
    lidx = _localize(m2g_dst, _NCH, 4)
    aggr = _sc_scatter_chunked(msgs, lidx, zeros, _NCH, 4)

    (p1, pb1), (pw2, pb2) = params["param_map"]
    mean, std = _m2g_final(grid_static, aggr, wnr, wna, bn1, wn2,
                           bn2, p1, pb1, pw2[:, :17], pb2[:17],
                           pw2[:, 17:], pb2[17:])
    return (mean[None], std[None])


# speedup vs baseline: 2.0396x; 2.0396x over previous
import functools

import jax
import jax.numpy as jnp
from jax import lax
from jax.experimental import pallas as pl
from jax.experimental.pallas import tpu as pltpu
from jax.experimental.pallas import tpu_sc as plsc

F32 = jnp.float32
_NC, _NS, _NW, _L = 2, 16, 32, 16
_C = 400
_CS = 200
_NCH = 12800
_ZR = 832


def _mesh():
    return plsc.VectorSubcoreMesh(core_axis_name="c", subcore_axis_name="s")


def _sc_gather(table, idx):
    e_tot = idx.shape[0]
    n_chunks = e_tot // _C
    iters = -(-n_chunks // _NW)

    @functools.partial(
        pl.kernel,
        out_type=jax.ShapeDtypeStruct((e_tot, 128), F32),
        mesh=_mesh(),
        scratch_types=[
            pltpu.VMEM((_C,), jnp.int32),
            pltpu.VMEM((_C,), jnp.int32),
            pltpu.VMEM((_C, 128), F32),
            pltpu.VMEM((_C, 128), F32),
            pltpu.SemaphoreType.DMA,
            pltpu.SemaphoreType.DMA,
            pltpu.SemaphoreType.DMA,
            pltpu.SemaphoreType.DMA,
        ],
    )
    def k(table_hbm, idx_hbm, out_hbm, iv0, iv1, rv0, rv1, sg0, sg1,
          so0, so1):
        iv = (iv0, iv1)
        rv = (rv0, rv1)
        wid = lax.axis_index("s") * _NC + lax.axis_index("c")
        sg = (sg0, sg1)
        so = (so0, so1)
        for j in range(iters):
            g = j * _NW + wid
            b = j % 2
            if j >= 2:
                gp = (j - 2) * _NW + wid

                @pl.when(gp < n_chunks)
                def _():
                    pltpu.make_async_copy(
                        rv[b], out_hbm.at[pl.ds(gp * _C, _C)],
                        so[b]).wait()

            @pl.when(g < n_chunks)
            def _():
                base = g * _C
                pltpu.sync_copy(idx_hbm.at[pl.ds(base, _C)], iv[b])
                pltpu.async_copy(table_hbm.at[iv[b]], rv[b],
                                 sg[b]).wait()
                pltpu.async_copy(rv[b], out_hbm.at[pl.ds(base, _C)],
                                 so[b])

        for j in range(max(0, iters - 2), iters):
            g = j * _NW + wid
            b = j % 2

            @pl.when(g < n_chunks)
            def _():
                pltpu.make_async_copy(
                    rv[b], out_hbm.at[pl.ds(g * _C, _C)], so[b]).wait()

    return k(table, idx)


def _sc_scatter2(msgs, idx, n_pad, zeros):
    e_tot = msgs.shape[0]
    n_chunks = e_tot // _C
    iters = -(-n_chunks // _NW)
    share = n_pad // _NS

    @functools.partial(
        pl.kernel,
        out_type=jax.ShapeDtypeStruct((2, n_pad, 128), F32),
        mesh=_mesh(),
        scratch_types=[
            pltpu.VMEM((_C,), jnp.int32),
            pltpu.VMEM((_C,), jnp.int32),
            pltpu.VMEM((_C, 128), F32),
            pltpu.VMEM((_C, 128), F32),
            pltpu.VMEM_SHARED((n_pad, 128), F32),
            pltpu.SemaphoreType.DMA,
            pltpu.SemaphoreType.DMA,
            pltpu.SemaphoreType.DMA,
            pltpu.SemaphoreType.DMA,
        ],
    )
    def k(msgs_hbm, idx_hbm, zeros_hbm, out_hbm, iv0, iv1, rv0, rv1, acc,
          sm0, sm1, ss0, ss1):
        iv = (iv0, iv1)
        rv = (rv0, rv1)
        cid = lax.axis_index("c")
        sid = lax.axis_index("s")
        wid = sid * _NC + cid
        sm = (sm0, sm1)
        ss = (ss0, ss1)
        pltpu.sync_copy(zeros_hbm.at[pl.ds(0, share)],
                        acc.at[pl.ds(sid * share, share)])
        plsc.subcore_barrier()
        for j in range(iters):
            g = j * _NW + wid
            b = j % 2
            if j >= 2:

                @pl.when((j - 2) * _NW + wid < n_chunks)
                def _():
                    pltpu.make_async_copy(rv[b], acc.at[iv[b]],
                                          ss[b]).wait()

            @pl.when(g < n_chunks)
            def _():
                base = g * _C
                pltpu.sync_copy(idx_hbm.at[pl.ds(base, _C)], iv[b])
                pltpu.async_copy(msgs_hbm.at[pl.ds(base, _C)], rv[b],
                                 sm[b]).wait()
                pltpu.async_copy(rv[b], acc.at[iv[b]], ss[b],
                                 add=True)

        for j in range(max(0, iters - 2), iters):
            b = j % 2

            @pl.when(j * _NW + wid < n_chunks)
            def _():
                pltpu.make_async_copy(rv[b], acc.at[iv[b]],
                                      ss[b]).wait()

        plsc.subcore_barrier()
        pltpu.sync_copy(acc.at[pl.ds(sid * share, share)],
                        out_hbm.at[cid, pl.ds(sid * share, share)])

    return k(msgs, idx, zeros)


def _sc_scatter_chunked(msgs, lidx, zeros, nch, nq):
    e_tot = msgs.shape[0]
    n_chunks = e_tot // _CS
    iters = -(-n_chunks // _NS)
    acc_rows = nch + 128
    share_z = acc_rows // _NS
    share_w = nch // _NS

    @functools.partial(
        pl.kernel,
        out_type=jax.ShapeDtypeStruct((nq * nch, 128), F32),
        mesh=_mesh(),
        scratch_types=[
            pltpu.VMEM((_CS,), jnp.int32),
            pltpu.VMEM((_CS,), jnp.int32),
            pltpu.VMEM((_CS, 128), F32),
            pltpu.VMEM_SHARED((acc_rows, 128), F32),
            pltpu.SemaphoreType.DMA,
            pltpu.SemaphoreType.DMA,
            pltpu.SemaphoreType.DMA,
            pltpu.SemaphoreType.DMA,
        ],
    )
    def k(msgs_hbm, lidx_hbm, zeros_hbm, out_hbm, iv0, iv1, rv0, acc,
          sm0, sm1, ss0, ss1):
        iv = (iv0, iv1)
        rv = (rv0, rv0)
        cid = lax.axis_index("c")
        sid = lax.axis_index("s")
        sm = (sm0, sm1)
        ss = (ss0, ss1)
        for p in range(nq // 2):
            chunk_id = 2 * p + cid
            pltpu.sync_copy(zeros_hbm.at[pl.ds(0, share_z)],
                            acc.at[pl.ds(sid * share_z, share_z)])
            plsc.subcore_barrier()
            for j in range(iters):
                g = j * _NS + sid
                b = j % 2
                nb = (j + 1) % 2
                if j >= 1:

                    @pl.when((j - 1) * _NS + sid < n_chunks)
                    def _():
                        pltpu.make_async_copy(
                            rv[nb], acc.at[iv[nb]], ss[nb]).wait()

                @pl.when(g < n_chunks)
                def _():
                    base = g * _CS
                    pltpu.sync_copy(
                        lidx_hbm.at[pl.ds(chunk_id * e_tot + base, _CS)],
                        iv[b])
                    pltpu.async_copy(msgs_hbm.at[pl.ds(base, _CS)],
                                     rv[b], sm[b]).wait()
                    pltpu.async_copy(rv[b], acc.at[iv[b]],
                                     ss[b], add=True)

            jl = iters - 1
            bl = jl % 2

            @pl.when(jl * _NS + sid < n_chunks)
            def _():
                pltpu.make_async_copy(
                    rv[bl], acc.at[iv[bl]], ss[bl]).wait()

            plsc.subcore_barrier()
            pltpu.sync_copy(
                acc.at[pl.ds(sid * share_w, share_w)],
                out_hbm.at[pl.ds(chunk_id * nch + sid * share_w, share_w)])
            plsc.subcore_barrier()

    return k(msgs, lidx, zeros)


def _pick_b(n):
    for b in (2000, 640):
        if n % b == 0:
            return b
    raise ValueError(f"no row block for {n}")


def _rowspec(b):
    return pl.BlockSpec((b, 128), lambda i: (i, 0))


def _wspec(shape):
    return pl.BlockSpec(shape, lambda i: (0,) * len(shape))


def _ln(y):
    m = jnp.mean(y, axis=-1, keepdims=True)
    d = y - m
    v = jnp.mean(d * d, axis=-1, keepdims=True)
    return d * lax.rsqrt(v + 1e-5)


def _mm(x, w, b=None):
    n = x.shape[0]
    blk = _pick_b(n)
    has_b = b is not None

    def body(x_ref, w_ref, *rest):
        if has_b:
            b_ref, o_ref = rest
        else:
            (o_ref,) = rest
        acc = jnp.dot(x_ref[...], w_ref[...], preferred_element_type=F32)
        if has_b:
            acc = acc + b_ref[...]
        o_ref[...] = acc

    args = [x, w] + ([b.reshape(1, 128)] if has_b else [])
    specs = [_rowspec(blk), _wspec((128, 128))] + ([_wspec((1, 128))] if has_b else [])
    return pl.pallas_call(
        body, grid=(n // blk,), in_specs=specs, out_specs=_rowspec(blk),
        out_shape=jax.ShapeDtypeStruct((n, 128), F32))(*args)


def _mlp2_ln(x, w1, b1, w2, b2):
    n = x.shape[0]
    blk = _pick_b(n)

    def body(x_ref, w1_ref, b1_ref, w2_ref, b2_ref, o_ref):
        h = jnp.dot(x_ref[...], w1_ref[...], preferred_element_type=F32) + b1_ref[...]
        h = jax.nn.silu(h)
        y = jnp.dot(h, w2_ref[...], preferred_element_type=F32) + b2_ref[...]
        o_ref[...] = _ln(y)

    return pl.pallas_call(
        body, grid=(n // blk,),
        in_specs=[_rowspec(blk), _wspec((128, 128)), _wspec((1, 128)),
                  _wspec((128, 128)), _wspec((1, 128))],
        out_specs=_rowspec(blk),
        out_shape=jax.ShapeDtypeStruct((n, 128), F32),
    )(x, w1, b1.reshape(1, 128), w2, b2.reshape(1, 128))


def _edge_msgs(edge, gs, gr, we, w2, b2, update_edges):
    e_tot = edge.shape[0]
    blk = _pick_b(e_tot)

    def body(e_ref, gs_ref, gr_ref, we_ref, w2_ref, b2_ref, m_ref, *ne):
        h = (jnp.dot(e_ref[...], we_ref[...], preferred_element_type=F32)
             + gs_ref[...] + gr_ref[...])
        h = jax.nn.silu(h)
        y = jnp.dot(h, w2_ref[...], preferred_element_type=F32) + b2_ref[...]
        msg = _ln(y)
        m_ref[...] = msg
        if update_edges:
            ne[0][...] = e_ref[...] + msg

    out_shape = [jax.ShapeDtypeStruct((e_tot, 128), F32)]
    out_specs = [_rowspec(blk)]
    if update_edges:
        out_shape.append(jax.ShapeDtypeStruct((e_tot, 128), F32))
        out_specs.append(_rowspec(blk))
    res = pl.pallas_call(
        body, grid=(e_tot // blk,),
        in_specs=[_rowspec(blk), _rowspec(blk), _rowspec(blk),
                  _wspec((128, 128)), _wspec((128, 128)), _wspec((1, 128))],
        out_specs=out_specs, out_shape=out_shape,
    )(edge, gs, gr, we, w2, b2.reshape(1, 128))
    return res if update_edges else (res[0], None)


def _node_update(rec, agg, wnr, wna, bn1, wn2, bn2, base_is_aggr, skip):
    n = rec.shape[0]
    blk = _pick_b(n)
    has_skip = skip is not None
    two_part = isinstance(agg, (tuple, list))

    def body(rec_ref, *refs):
        refs = list(refs)
        a0_ref = refs.pop(0)
        a1_ref = refs.pop(0) if two_part else None
        wnr_ref, wna_ref, bn1_ref, wn2_ref, bn2_ref = refs[:5]
        rest = refs[5:]
        if has_skip:
            skip_ref, o_ref = rest
        else:
            (o_ref,) = rest
        aggr = a0_ref[...]
        if two_part:
            aggr = aggr + a1_ref[...]
        h = (jnp.dot(rec_ref[...], wnr_ref[...], preferred_element_type=F32)
             + jnp.dot(aggr, wna_ref[...], preferred_element_type=F32)
             + bn1_ref[...])
        h = jax.nn.silu(h)
        y = jnp.dot(h, wn2_ref[...], preferred_element_type=F32) + bn2_ref[...]
        upd = _ln(y)
        out = (aggr if base_is_aggr else rec_ref[...]) + upd
        if has_skip:
            out = out + skip_ref[...]
        o_ref[...] = out

    aggs = list(agg) if two_part else [agg]
    args = [rec] + aggs + [wnr, wna, bn1.reshape(1, 128), wn2,
                           bn2.reshape(1, 128)]
    specs = ([_rowspec(blk)] + [_rowspec(blk)] * len(aggs)
             + [_wspec((128, 128)), _wspec((128, 128)), _wspec((1, 128)),
                _wspec((128, 128)), _wspec((1, 128))])
    if has_skip:
        args.append(skip)
        specs.append(_rowspec(blk))
    return pl.pallas_call(
        body, grid=(n // blk,), in_specs=specs, out_specs=_rowspec(blk),
        out_shape=jax.ShapeDtypeStruct((n, 128), F32))(*args)


def _localize(dst, nch, nq):
    e_tot = dst.shape[0]
    blk = 2000
    d2 = dst.reshape(e_tot // blk, 1, blk)

    def body(d_ref, o_ref):
        v = d_ref[...]
        for q in range(nq):
            t = v - q * nch
            ok = (t >= 0) & (t < nch)
            o_ref[q, ...] = jnp.where(ok, t, nch)

    out = pl.pallas_call(
        body, grid=(e_tot // blk,),
        in_specs=[pl.BlockSpec((1, 1, blk), lambda i: (i, 0, 0))],
        out_specs=pl.BlockSpec((nq, 1, 1, blk), lambda i: (0, i, 0, 0)),
        out_shape=jax.ShapeDtypeStruct((nq, e_tot // blk, 1, blk), jnp.int32),
    )(d2)
    return out.reshape(nq * e_tot)


def _softplus(x):
    return jnp.maximum(x, 0.0) + jnp.log1p(jnp.exp(-jnp.abs(x)))


def _m2g_final(rec, aggr, wnr, wna, bn1, wn2, bn2, p1, pb1, w_mean, b_mean,
               w_std, b_std):
    n = rec.shape[0]
    blk = _pick_b(n)

    def body(rec_ref, a_ref, wnr_ref, wna_ref, bn1_ref, wn2_ref, bn2_ref,
             p1_ref, pb1_ref, wm_ref, bm_ref, ws_ref, bs_ref,
             mean_ref, std_ref):
        aggr = a_ref[...]
        h = (jnp.dot(rec_ref[...], wnr_ref[...], preferred_element_type=F32)
             + jnp.dot(aggr, wna_ref[...], preferred_element_type=F32)
             + bn1_ref[...])
        h = jax.nn.silu(h)
        y = jnp.dot(h, wn2_ref[...], preferred_element_type=F32) + bn2_ref[...]
        grid_rep = aggr + _ln(y)
        g = jax.nn.silu(
            jnp.dot(grid_rep, p1_ref[...], preferred_element_type=F32)
            + pb1_ref[...])
        mean_ref[...] = (jnp.dot(g, wm_ref[...], preferred_element_type=F32)
                         + bm_ref[...])
        std_ref[...] = _softplus(
            jnp.dot(g, ws_ref[...], preferred_element_type=F32) + bs_ref[...])

    ospec = pl.BlockSpec((blk, 17), lambda i: (i, 0))
    return pl.pallas_call(
        body, grid=(n // blk,),
        in_specs=[_rowspec(blk), _rowspec(blk), _wspec((128, 128)),
                  _wspec((128, 128)), _wspec((1, 128)), _wspec((128, 128)),
                  _wspec((1, 128)), _wspec((128, 128)), _wspec((1, 128)),
                  _wspec((128, 17)), _wspec((1, 17)), _wspec((128, 17)),
                  _wspec((1, 17))],
        out_specs=[ospec, ospec],
        out_shape=[jax.ShapeDtypeStruct((n, 17), F32),
                   jax.ShapeDtypeStruct((n, 17), F32)],
    )(rec, aggr, wnr, wna, bn1.reshape(1, 128), wn2, bn2.reshape(1, 128),
      p1, pb1.reshape(1, 128), w_mean, b_mean.reshape(1, 17), w_std,
      b_std.reshape(1, 17))


def _padn(x, n):
    return jnp.pad(x, ((0, n - x.shape[0]), (0, 0)))


def _split_edge_w(ip):
    (w1, b1), (w2, b2) = ip["edge"]
    return w1[:128], w1[128:256], w1[256:384], b1, w2, b2


def _split_node_w(ip):
    (wn1, bn1), (wn2, bn2) = ip["node"]
    return wn1[:128], wn1[128:], bn1, wn2, bn2


def _stage(x_send, x_rec, edge, src, dst, ip, n_pad, zeros, *,
           base_is_aggr, skip, update_edges):
    we, ws, wr, b1, w2, b2 = _split_edge_w(ip)
    wnr, wna, bn1, wn2, bn2 = _split_node_w(ip)
    ts = _mm(x_send, ws)
    tr = _mm(x_rec, wr, b1)
    gs = _sc_gather(ts, src)
    gr = _sc_gather(tr, dst)
    msgs, new_edge = _edge_msgs(edge, gs, gr, we, w2, b2, update_edges)
    if n_pad <= 2560:
        agg = _sc_scatter2(msgs, dst, n_pad, zeros)
        agg = (agg[0], agg[1])
    else:
        nch = n_pad // 2
        lidx = _localize(dst, nch, 2)
        agg = _sc_scatter_chunked(msgs, lidx, zeros, nch, 2)
    x_new = _node_update(x_rec, agg, wnr, wna, bn1, wn2, bn2,
                         base_is_aggr, skip)
    return x_new, new_edge


def kernel(latent_samples, skip_in_0, skip_in_1, skip_in_2, skip_up_0,
           skip_up_1, mesh_emb_0, mesh_emb_1, mesh_emb_2, mesh_down_emb_0,
           mesh_down_emb_1, m2m_emb_0, m2m_emb_1, m2m_emb_2, m2g_emb,
           grid_static_features_hr, params, m2g_src, m2g_dst, md0_src,
           md0_dst, md1_src, md1_dst, m2m0, m2m1):
    zeros = jnp.zeros((_ZR, 128), F32)

    lat = _padn(latent_samples[0], 640)
    mesh1 = _padn(mesh_emb_1[0], 2560)
    mesh0 = _padn(mesh_emb_0[0], 10240)
    sk_up0 = _padn(skip_up_0[0], 2560)
    sk_in1 = _padn(skip_in_1[0], 2560)
    sk_in0 = _padn(skip_in_0[0], 10240)
    grid_static = grid_static_features_hr[0]

    (w1, b1), (w2, b2) = params["latent_embedder"]
    rep2 = _mlp2_ln(lat, w1, b1, w2, b2)

    rep1, _ = _stage(rep2, mesh1, mesh_down_emb_1[0], md1_src, md1_dst,
                     params["mesh_down"][1], 2560, zeros,
                     base_is_aggr=True, skip=sk_up0, update_edges=False)

    x, e = rep1, m2m_emb_1[0]
    x, e = _stage(x, x, e, m2m1[0], m2m1[1], params["intra_down"][1][0],
                  2560, zeros, base_is_aggr=False, skip=None,
                  update_edges=True)
    x, _ = _stage(x, x, e, m2m1[0], m2m1[1], params["intra_down"][1][1],
                  2560, zeros, base_is_aggr=False, skip=sk_in1,
                  update_edges=False)

    mesh_new0, _ = _stage(x, mesh0, mesh_down_emb_0[0], md0_src, md0_dst,
                          params["mesh_down"][0], 10240, zeros,
                          base_is_aggr=True, skip=None, update_edges=False)

    x, e = mesh_new0, m2m_emb_0[0]
    x, e = _stage(x, x, e, m2m0[0], m2m0[1], params["intra_down"][0][0],
                  10240, zeros, base_is_aggr=False, skip=None,
                  update_edges=True)
    x, _ = _stage(x, x, e, m2m0[0], m2m0[1], params["intra_down"][0][1],
                  10240, zeros, base_is_aggr=False, skip=sk_in0,
                  update_edges=False)

    ip = params["m2g"]
    we, ws, wr, eb1, w2, b2 = _split_edge_w(ip)
    wnr, wna, bn1, wn2, bn2 = _split_node_w(ip)
    ts = _mm(x, ws)
    tr = _mm(grid_static, wr, eb1)
    gs = _sc_gather(ts, m2g_src)
    gr = _sc_gather(tr, m2g_dst)
    msgs, _ = _edge_msgs(m2g_emb[0], gs, gr, we, w2, b2, False)
    lidx = _localize(m2g_dst, _NCH, 4)
    aggr = _sc_scatter_chunked(msgs, lidx, zeros, _NCH, 4)

    (p1, pb1), (pw2, pb2) = params["param_map"]
    mean, std = _m2g_final(grid_static, aggr, wnr, wna, bn1, wn2,
                           bn2, p1, pb1, pw2[:, :17], pb2[:17],
                           pw2[:, 17:], pb2[17:])
    return (mean[None], std[None])
